# Initial kernel scaffold; baseline (speedup 1.0000x reference)
import jax
import jax.numpy as jnp
from jax.experimental import pallas as pl


def kernel(inputs, token_table, position_table):
    raise NotImplementedError("write your pallas kernel here")


# speedup vs baseline: 3.0612x; 3.0612x over previous
import functools

import jax
import jax.numpy as jnp
from jax import lax
from jax.experimental import pallas as pl
from jax.experimental.pallas import tpu as pltpu
from jax.experimental.pallas import tpu_sc as plsc

SEQ_LEN = 200
EMBED = 64
NUM_CORES = 2
NUM_SUBCORES = 16
NUM_WORKERS = NUM_CORES * NUM_SUBCORES


def _sc_body(idx_hbm, tok_hbm, pos_hbm, out_hbm, pos_v, idx_v, rows_v, sem):
    wid = lax.axis_index("s") * NUM_CORES + lax.axis_index("c")
    batch = idx_hbm.shape[0]
    seqs_per_w = batch // NUM_WORKERS

    pltpu.sync_copy(pos_hbm, pos_v)

    def seq_body(i, carry):
        seq = wid * seqs_per_w + i
        pltpu.sync_copy(idx_hbm.at[seq], idx_v)
        cp0 = pltpu.async_copy(
            tok_hbm.at[idx_v.at[0]], rows_v.at[pl.ds(0, 100)], sem
        )
        cp1 = pltpu.async_copy(
            tok_hbm.at[idx_v.at[1]], rows_v.at[pl.ds(100, 100)], sem
        )
        cp0.wait()
        cp1.wait()

        def row_body(r, c):
            for e in range(EMBED // 16):
                sl = pl.ds(e * 16, 16)
                rows_v[r, sl] = rows_v[r, sl] + pos_v[r, sl]
            return c

        lax.fori_loop(0, SEQ_LEN, row_body, 0)

        pltpu.sync_copy(rows_v, out_hbm.at[seq])
        return carry

    lax.fori_loop(0, seqs_per_w, seq_body, 0)


def kernel(inputs, token_table, position_table):
    batch = inputs.shape[0]
    idx = inputs.astype(jnp.int32).reshape(batch, 2, SEQ_LEN // 2)

    mesh = plsc.VectorSubcoreMesh(core_axis_name="c", subcore_axis_name="s")
    k = functools.partial(
        pl.kernel,
        out_type=jax.ShapeDtypeStruct((batch, SEQ_LEN, EMBED), jnp.float32),
        mesh=mesh,
        compiler_params=pltpu.CompilerParams(use_tc_tiling_on_sc=False),
        scratch_types=[
            pltpu.VMEM((SEQ_LEN, EMBED), jnp.float32),
            pltpu.VMEM((2, SEQ_LEN // 2), jnp.int32),
            pltpu.VMEM((SEQ_LEN, EMBED), jnp.float32),
            pltpu.SemaphoreType.DMA,
        ],
    )(_sc_body)
    return k(idx, token_table, position_table)


# speedup vs baseline: 4.1528x; 1.3566x over previous
import functools

import jax
import jax.numpy as jnp
from jax import lax
from jax.experimental import pallas as pl
from jax.experimental.pallas import tpu as pltpu
from jax.experimental.pallas import tpu_sc as plsc

SEQ_LEN = 200
HALF = SEQ_LEN // 2
EMBED = 64
NUM_CORES = 2
NUM_SUBCORES = 16
NUM_WORKERS = NUM_CORES * NUM_SUBCORES
NBUF = 4


def _sc_body(idx_hbm, tok_hbm, pos_hbm, out_hbm, pos_v, idx_v, rows, gsems, ssems):
    wid = lax.axis_index("s") * NUM_CORES + lax.axis_index("c")
    batch = idx_hbm.shape[0]
    seqs_per_w = batch // NUM_WORKERS
    seq0 = wid * seqs_per_w

    pltpu.sync_copy(idx_hbm.at[pl.ds(seq0, seqs_per_w)], idx_v)
    pltpu.sync_copy(pos_hbm, pos_v)

    def fire_gather(c, b):
        for k in range(2):
            pltpu.async_copy(
                tok_hbm.at[idx_v.at[c].at[k]],
                rows.at[b].at[pl.ds(k * HALF, HALF)],
                gsems.at[b],
            )

    def drain_gather(b):
        pltpu.make_async_copy(
            tok_hbm.at[pl.ds(0, SEQ_LEN)], rows.at[b], gsems.at[b]
        ).wait()

    def fire_scatter(c, b):
        pltpu.async_copy(rows.at[b], out_hbm.at[seq0 + c], ssems.at[b])

    def drain_scatter(b):
        pltpu.make_async_copy(
            rows.at[b], out_hbm.at[0], ssems.at[b]
        ).wait()

    def add_pos(b):
        def row_body(r, carry):
            for e in range(EMBED // 16):
                sl = pl.ds(e * 16, 16)
                rows[b, r, sl] = rows[b, r, sl] + pos_v[r, sl]
            return carry

        lax.fori_loop(0, SEQ_LEN, row_body, 0)

    for b in range(NBUF - 1):
        fire_gather(b, b)

    def body(i, carry):
        for j in range(NBUF):
            c = i * NBUF + j
            drain_gather(j)
            add_pos(j)
            fire_scatter(c, j)
            pb = (j + NBUF - 1) % NBUF
            if j == 0:

                @pl.when(i > 0)
                def _():
                    drain_scatter(pb)

                fire_gather(c + NBUF - 1, pb)
            else:
                drain_scatter(pb)

                @pl.when(c + NBUF - 1 < seqs_per_w)
                def _():
                    fire_gather(c + NBUF - 1, pb)

        return carry

    lax.fori_loop(0, seqs_per_w // NBUF, body, 0)

    drain_scatter(NBUF - 1)


def kernel(inputs, token_table, position_table):
    batch = inputs.shape[0]
    idx = inputs.astype(jnp.int32).reshape(batch, 2, HALF)

    mesh = plsc.VectorSubcoreMesh(core_axis_name="c", subcore_axis_name="s")
    k = functools.partial(
        pl.kernel,
        out_type=jax.ShapeDtypeStruct((batch, SEQ_LEN, EMBED), jnp.float32),
        mesh=mesh,
        compiler_params=pltpu.CompilerParams(use_tc_tiling_on_sc=False),
        scratch_types=[
            pltpu.VMEM((SEQ_LEN, EMBED), jnp.float32),
            pltpu.VMEM((batch // NUM_WORKERS, 2, HALF), jnp.int32),
            pltpu.VMEM((NBUF, SEQ_LEN, EMBED), jnp.float32),
            pltpu.SemaphoreType.DMA((NBUF,)),
            pltpu.SemaphoreType.DMA((NBUF,)),
        ],
    )(_sc_body)
    return k(idx, token_table, position_table)


# speedup vs baseline: 4.1532x; 1.0001x over previous
import functools

import jax
import jax.numpy as jnp
from jax import lax
from jax.experimental import pallas as pl
from jax.experimental.pallas import tpu as pltpu
from jax.experimental.pallas import tpu_sc as plsc

SEQ_LEN = 200
HALF = SEQ_LEN // 2
EMBED = 64
NUM_CORES = 2
NUM_SUBCORES = 16
NUM_WORKERS = NUM_CORES * NUM_SUBCORES
NBUF = 4


def _sc_body(idx_hbm, tok_hbm, pos_hbm, out_hbm, pos_v, idx_v, rows, gsems, ssems):
    wid = lax.axis_index("s") * NUM_CORES + lax.axis_index("c")
    batch = idx_hbm.shape[0]
    seqs_per_w = batch // NUM_WORKERS
    seq0 = wid * seqs_per_w

    pltpu.sync_copy(idx_hbm.at[pl.ds(seq0, seqs_per_w)], idx_v)
    pltpu.sync_copy(pos_hbm, pos_v)

    def fire_gather(c, b):
        for k in range(2):
            pltpu.async_copy(
                tok_hbm.at[idx_v.at[c].at[k]],
                rows.at[b].at[pl.ds(k * HALF, HALF)],
                gsems.at[b],
            )

    def drain_gather(b):
        pltpu.make_async_copy(
            tok_hbm.at[pl.ds(0, SEQ_LEN)], rows.at[b], gsems.at[b]
        ).wait()

    def fire_scatter(c, b):
        pltpu.async_copy(
            rows.at[b], out_hbm.at[pl.ds((seq0 + c) * SEQ_LEN, SEQ_LEN)], ssems.at[b]
        )

    def drain_scatter(b):
        pltpu.make_async_copy(
            rows.at[b], out_hbm.at[pl.ds(0, SEQ_LEN)], ssems.at[b]
        ).wait()

    def add_pos(b):
        def row_body(r, carry):
            for e in range(EMBED // 16):
                sl = pl.ds(e * 16, 16)
                rows[b, r, sl] = rows[b, r, sl] + pos_v[r, sl]
            return carry

        lax.fori_loop(0, SEQ_LEN, row_body, 0)

    for b in range(NBUF - 1):
        fire_gather(b, b)

    def body(i, carry):
        for j in range(NBUF):
            c = i * NBUF + j
            drain_gather(j)
            add_pos(j)
            fire_scatter(c, j)
            pb = (j + NBUF - 1) % NBUF
            if j == 0:

                @pl.when(i > 0)
                def _():
                    drain_scatter(pb)

                fire_gather(c + NBUF - 1, pb)
            else:
                drain_scatter(pb)

                @pl.when(c + NBUF - 1 < seqs_per_w)
                def _():
                    fire_gather(c + NBUF - 1, pb)

        return carry

    lax.fori_loop(0, seqs_per_w // NBUF, body, 0)

    drain_scatter(NBUF - 1)


def kernel(inputs, token_table, position_table):
    batch = inputs.shape[0]
    idx = inputs.astype(jnp.int32).reshape(batch, 2, HALF)

    mesh = plsc.VectorSubcoreMesh(core_axis_name="c", subcore_axis_name="s")
    k = functools.partial(
        pl.kernel,
        out_type=jax.ShapeDtypeStruct((batch * SEQ_LEN, EMBED), jnp.float32),
        mesh=mesh,
        compiler_params=pltpu.CompilerParams(use_tc_tiling_on_sc=False),
        scratch_types=[
            pltpu.VMEM((SEQ_LEN, EMBED), jnp.float32),
            pltpu.VMEM((batch // NUM_WORKERS, 2, HALF), jnp.int32),
            pltpu.VMEM((NBUF, SEQ_LEN, EMBED), jnp.float32),
            pltpu.SemaphoreType.DMA((NBUF,)),
            pltpu.SemaphoreType.DMA((NBUF,)),
        ],
    )(_sc_body)
    return k(idx, token_table, position_table).reshape(batch, SEQ_LEN, EMBED)
